# baseline (device time: 29756 ns/iter reference)
import jax
import jax.numpy as jnp
from jax import lax
from jax.experimental import pallas as pl
from jax.experimental.pallas import tpu as pltpu

N_Y = 4
S = N_Y - 1
Q = 8


def kernel(x):
    m_per, n = x.shape
    m2 = m_per // 2
    mq = m2 // Q

    def body(
        x_ref, out_ref, own_buf, rs_buf, ls_buf, xr_buf, xl_buf,
        rs_ssem, rs_rsem, ls_ssem, ls_rsem,
        xr_ssem, xr_rsem, xl_ssem, xl_rsem,
    ):
        my_x = lax.axis_index("x")
        my_y = lax.axis_index("y")
        my_z = lax.axis_index("z")
        right = (my_x, jnp.minimum(my_y + 1, N_Y - 1), my_z)
        left = (my_x, jnp.maximum(my_y - 1, 0), my_z)
        peer = (1 - my_x, my_y, my_z)
        has_r = my_y < N_Y - 1
        has_l = my_y > 0
        edge = jnp.logical_or(my_y == 0, my_y == N_Y - 1)
        my_off = my_x * m2
        other_off = (1 - my_x) * m2

        def vs_rs(s):
            return jnp.logical_and(has_r, my_y >= s)

        def vr_rs(s):
            return my_y >= s + 1

        def vs_ls(s):
            return jnp.logical_and(has_l, my_y + s <= N_Y - 1)

        def vr_ls(s):
            return my_y + 1 + s <= N_Y - 1

        bar = pltpu.get_barrier_semaphore()

        @pl.when(has_r)
        def _():
            pl.semaphore_signal(
                bar, inc=1, device_id=right,
                device_id_type=pl.DeviceIdType.MESH,
            )

        @pl.when(has_l)
        def _():
            pl.semaphore_signal(
                bar, inc=1, device_id=left,
                device_id_type=pl.DeviceIdType.MESH,
            )

        pl.semaphore_signal(
            bar, inc=jnp.where(edge, 2, 1), device_id=peer,
            device_id_type=pl.DeviceIdType.MESH,
        )
        pl.semaphore_wait(bar, 3)

        def rcopy(src, dst, ssem, rsem, dev):
            return pltpu.make_async_remote_copy(
                src_ref=src, dst_ref=dst, send_sem=ssem, recv_sem=rsem,
                device_id=dev, device_id_type=pl.DeviceIdType.MESH,
            )

        def sub(ref, q):
            return ref.at[pl.ds(q * mq, mq), :]

        rs_d = [
            [rcopy(sub(own_buf, q) if s == 0 else rs_buf.at[s - 1, q],
                   rs_buf.at[s, q],
                   rs_ssem.at[s * Q + q], rs_rsem.at[s * Q + q], right)
             for q in range(Q)]
            for s in range(S)
        ]
        ls_d = [
            [rcopy(sub(own_buf, q) if s == 0 else ls_buf.at[s - 1, q],
                   ls_buf.at[s, q],
                   ls_ssem.at[s * Q + q], ls_rsem.at[s * Q + q], left)
             for q in range(Q)]
            for s in range(S)
        ]
        xr_d = [
            [rcopy(rs_buf.at[s, q], xr_buf.at[s, q],
                   xr_ssem.at[s * Q + q], xr_rsem.at[s * Q + q], peer)
             for q in range(Q)]
            for s in range(S)
        ]
        xl_d = [
            [rcopy(ls_buf.at[s, q], xl_buf.at[s, q],
                   xl_ssem.at[s * Q + q], xl_rsem.at[s * Q + q], peer)
             for q in range(Q)]
            for s in range(S)
        ]

        for q in range(Q):
            own_buf[pl.ds(q * mq, mq), :] = x_ref[
                pl.ds(my_off + q * mq, mq), :
            ]

            @pl.when(vs_rs(0))
            def _(q=q):
                rs_d[0][q].start()

            @pl.when(vs_ls(0))
            def _(q=q):
                ls_d[0][q].start()

        for s in range(S):
            for q in range(Q):
                @pl.when(vr_rs(s))
                def _(s=s, q=q):
                    rs_d[s][q].wait_recv()

                if s + 1 < S:
                    @pl.when(vs_rs(s + 1))
                    def _(s=s, q=q):
                        rs_d[s + 1][q].start()

                @pl.when(vr_rs(s))
                def _(s=s, q=q):
                    xr_d[s][q].start()

                @pl.when(vr_ls(s))
                def _(s=s, q=q):
                    ls_d[s][q].wait_recv()

                if s + 1 < S:
                    @pl.when(vs_ls(s + 1))
                    def _(s=s, q=q):
                        ls_d[s + 1][q].start()

                @pl.when(vr_ls(s))
                def _(s=s, q=q):
                    xl_d[s][q].start()

        out_ref[pl.ds(my_y * m_per, m_per), :] = x_ref[...]
        for s in range(S):
            @pl.when(vr_rs(s))
            def _(s=s):
                c = jnp.clip(my_y - 1 - s, 0, N_Y - 1)
                out_ref[pl.ds(c * m_per + my_off, m2), :] = (
                    rs_buf[s].reshape(m2, n)
                )

            @pl.when(vr_ls(s))
            def _(s=s):
                c = jnp.clip(my_y + 1 + s, 0, N_Y - 1)
                out_ref[pl.ds(c * m_per + my_off, m2), :] = (
                    ls_buf[s].reshape(m2, n)
                )

        for s in range(S):
            @pl.when(vr_rs(s))
            def _(s=s):
                for q in range(Q):
                    xr_d[s][q].wait_recv()
                c = jnp.clip(my_y - 1 - s, 0, N_Y - 1)
                out_ref[pl.ds(c * m_per + other_off, m2), :] = (
                    xr_buf[s].reshape(m2, n)
                )

            @pl.when(vr_ls(s))
            def _(s=s):
                for q in range(Q):
                    xl_d[s][q].wait_recv()
                c = jnp.clip(my_y + 1 + s, 0, N_Y - 1)
                out_ref[pl.ds(c * m_per + other_off, m2), :] = (
                    xl_buf[s].reshape(m2, n)
                )

        for s in range(S):
            for q in range(Q):
                @pl.when(vs_rs(s))
                def _(s=s, q=q):
                    rs_d[s][q].wait_send()

                @pl.when(vs_ls(s))
                def _(s=s, q=q):
                    ls_d[s][q].wait_send()

                @pl.when(vr_rs(s))
                def _(s=s, q=q):
                    xr_d[s][q].wait_send()

                @pl.when(vr_ls(s))
                def _(s=s, q=q):
                    xl_d[s][q].wait_send()

    return pl.pallas_call(
        body,
        out_shape=jax.ShapeDtypeStruct((N_Y * m_per, n), x.dtype),
        in_specs=[pl.BlockSpec(memory_space=pltpu.VMEM)],
        out_specs=pl.BlockSpec(memory_space=pltpu.VMEM),
        scratch_shapes=[
            pltpu.VMEM((m2, n), x.dtype),
            pltpu.VMEM((S, Q, mq, n), x.dtype),
            pltpu.VMEM((S, Q, mq, n), x.dtype),
            pltpu.VMEM((S, Q, mq, n), x.dtype),
            pltpu.VMEM((S, Q, mq, n), x.dtype),
            pltpu.SemaphoreType.DMA((S * Q,)),
            pltpu.SemaphoreType.DMA((S * Q,)),
            pltpu.SemaphoreType.DMA((S * Q,)),
            pltpu.SemaphoreType.DMA((S * Q,)),
            pltpu.SemaphoreType.DMA((S * Q,)),
            pltpu.SemaphoreType.DMA((S * Q,)),
            pltpu.SemaphoreType.DMA((S * Q,)),
            pltpu.SemaphoreType.DMA((S * Q,)),
        ],
        compiler_params=pltpu.CompilerParams(collective_id=0),
    )(x)
